# Initial kernel scaffold; baseline (speedup 1.0000x reference)
import jax
import jax.numpy as jnp
from jax.experimental import pallas as pl


def kernel(features, edge_index, W1, b1, W2, b2):
    raise NotImplementedError("write your pallas kernel here")


# speedup vs baseline: 8.4579x; 8.4579x over previous
import functools

import jax
import jax.numpy as jnp
from jax import lax
from jax.experimental import pallas as pl
from jax.experimental.pallas import tpu as pltpu
from jax.experimental.pallas import tpu_sc as plsc

N_NODES = 100000
N_EDGES = 3200000
D_PAD = 32

NC = 2
NS = 16

R_PER_SC = N_NODES // NC
ACC_ROWS = 51200
E_PER_TILE = N_EDGES // NS
OUTER = 2000
CH = 80
N_OUTER = E_PER_TILE // OUTER
N_INNER = OUTER // CH
ZCH = 80
OUT_CH = 1000
N_OUT_CHUNKS = R_PER_SC // OUT_CH


def _sc_aggregate(table_pad, src_idx, dst_idx):
    mesh = plsc.VectorSubcoreMesh(core_axis_name="c", subcore_axis_name="s")

    @functools.partial(
        pl.kernel,
        out_type=jax.ShapeDtypeStruct((N_NODES, D_PAD), jnp.float32),
        mesh=mesh,
        scratch_types=[
            pltpu.VMEM_SHARED((ACC_ROWS, D_PAD), jnp.float32),
            pltpu.VMEM((OUTER,), jnp.int32),
            pltpu.VMEM((OUTER,), jnp.int32),
            pltpu.VMEM((CH,), jnp.int32),
            pltpu.VMEM((CH,), jnp.int32),
            pltpu.VMEM((CH, D_PAD), jnp.float32),
            pltpu.SemaphoreType.DMA,
        ],
        compiler_params=pltpu.CompilerParams(use_tc_tiling_on_sc=False),
    )
    def agg(table, sidx, didx, out, acc, src_big, dst_big, src_sm, dst_sm, rows, sem):
        c = lax.axis_index("c")
        s = lax.axis_index("s")
        base = c * R_PER_SC

        zero = jnp.zeros((16,), jnp.float32)
        for r in range(ZCH):
            for q in range(D_PAD // 16):
                rows[r, pl.ds(q * 16, 16)] = zero
        z0 = s * (ACC_ROWS // NS)

        def zloop(j, _):
            pltpu.sync_copy(rows, acc.at[pl.ds(z0 + j * ZCH, ZCH)])
            return 0

        lax.fori_loop(0, (ACC_ROWS // NS) // ZCH, zloop, 0)
        plsc.subcore_barrier()

        e0 = s * E_PER_TILE

        def outer(j, _):
            off = e0 + j * OUTER
            pltpu.sync_copy(sidx.at[pl.ds(off, OUTER)], src_big)
            pltpu.sync_copy(didx.at[pl.ds(off, OUTER)], dst_big)

            def inner(k, _):
                kb = k * CH
                for v in range(CH // 16):
                    t = dst_big[pl.ds(kb + v * 16, 16)]
                    m = (t >= base) & (t < base + R_PER_SC)
                    dst_sm[pl.ds(v * 16, 16)] = jnp.where(m, t - base, R_PER_SC)
                    src_sm[pl.ds(v * 16, 16)] = src_big[pl.ds(kb + v * 16, 16)]
                pltpu.async_copy(table.at[src_sm], rows, sem).wait()
                pltpu.sync_copy(rows, acc.at[dst_sm], add=True)
                return 0

            lax.fori_loop(0, N_INNER, inner, 0)
            return 0

        lax.fori_loop(0, N_OUTER, outer, 0)
        plsc.subcore_barrier()

        n_chunks = (N_OUT_CHUNKS - s + NS - 1) // NS

        def cloop(i, _):
            r0 = (s + i * NS) * OUT_CH
            pltpu.sync_copy(
                acc.at[pl.ds(r0, OUT_CH)], out.at[pl.ds(base + r0, OUT_CH)]
            )
            return 0

        lax.fori_loop(0, n_chunks, cloop, 0)

    return agg(table_pad, src_idx, dst_idx)


_BN = 2000
_NBLK = N_NODES // _BN


def _mlp_body(h_ref, w1_ref, b1_ref, w2_ref, o_ref):
    t = jnp.dot(h_ref[...], w1_ref[...], preferred_element_type=jnp.float32)
    t = jnp.maximum(t + b1_ref[...], 0.0)
    o_ref[...] = jnp.dot(t, w2_ref[...], preferred_element_type=jnp.float32)


def _mlp(h1p, W1p, b1r, W2p):
    return pl.pallas_call(
        _mlp_body,
        grid=(_NBLK,),
        in_specs=[
            pl.BlockSpec((_BN, D_PAD), lambda i: (i, 0)),
            pl.BlockSpec((D_PAD, 48), lambda i: (0, 0)),
            pl.BlockSpec((1, 48), lambda i: (0, 0)),
            pl.BlockSpec((48, D_PAD), lambda i: (0, 0)),
        ],
        out_specs=pl.BlockSpec((_BN, D_PAD), lambda i: (i, 0)),
        out_shape=jax.ShapeDtypeStruct((N_NODES, D_PAD), jnp.float32),
    )(h1p, W1p, b1r, W2p)


def _stats_body(h_ref, b2_ref, sum_ref, sq_ref):
    i = pl.program_id(0)
    x = h_ref[...] + b2_ref[...]
    col = lax.broadcasted_iota(jnp.int32, (_BN, D_PAD), 1)
    xs = jnp.where(col < 24, x, 0.0)

    @pl.when(i == 0)
    def _():
        sum_ref[...] = jnp.zeros((1, 1), jnp.float32)
        sq_ref[...] = jnp.zeros((1, 1), jnp.float32)

    sum_ref[...] += jnp.full((1, 1), jnp.sum(xs), jnp.float32)
    sq_ref[...] += jnp.full((1, 1), jnp.sum(xs * xs), jnp.float32)


def _stats(h2p, b2r):
    return pl.pallas_call(
        _stats_body,
        grid=(_NBLK,),
        in_specs=[
            pl.BlockSpec((_BN, D_PAD), lambda i: (i, 0)),
            pl.BlockSpec((1, D_PAD), lambda i: (0, 0)),
        ],
        out_specs=[
            pl.BlockSpec((1, 1), lambda i: (0, 0)),
            pl.BlockSpec((1, 1), lambda i: (0, 0)),
        ],
        out_shape=[
            jax.ShapeDtypeStruct((1, 1), jnp.float32),
            jax.ShapeDtypeStruct((1, 1), jnp.float32),
        ],
    )(h2p, b2r)


def _apply_body(h_ref, b2_ref, mu_ref, inv_ref, o_ref):
    x = h_ref[...][:, :24] + b2_ref[...][:, :24]
    o_ref[...] = (x - mu_ref[0, 0]) * inv_ref[0, 0]


def _apply(h2p, b2r, mu, inv):
    return pl.pallas_call(
        _apply_body,
        grid=(_NBLK,),
        in_specs=[
            pl.BlockSpec((_BN, D_PAD), lambda i: (i, 0)),
            pl.BlockSpec((1, D_PAD), lambda i: (0, 0)),
            pl.BlockSpec((1, 1), lambda i: (0, 0)),
            pl.BlockSpec((1, 1), lambda i: (0, 0)),
        ],
        out_specs=pl.BlockSpec((_BN, 24), lambda i: (i, 0)),
        out_shape=jax.ShapeDtypeStruct((N_NODES, 24), jnp.float32),
    )(h2p, b2r, mu, inv)


def kernel(features, edge_index, W1, b1, W2, b2):
    n, d = features.shape
    assert n == N_NODES and d == 24 and edge_index.shape == (2, N_EDGES)

    fpad = jnp.pad(features, ((0, 0), (0, D_PAD - 24)))
    W1p = jnp.pad(W1, ((0, D_PAD - 24), (0, 0)))
    W2p = jnp.pad(W2, ((0, 0), (0, D_PAD - 24)))
    b1r = b1.reshape(1, 48)
    b2r = jnp.pad(b2, (0, D_PAD - 24)).reshape(1, D_PAD)

    src_idx = edge_index[0]
    dst_idx = edge_index[1]
    h1p = _sc_aggregate(fpad, src_idx, dst_idx)
    yp = _mlp(h1p, W1p, b1r, W2p)
    h2p = _sc_aggregate(yp, src_idx, dst_idx)

    ssum, ssq = _stats(h2p, b2r)
    cnt = jnp.float32(N_NODES * 24)
    mu = ssum / cnt
    var = (ssq - cnt * mu * mu) / (cnt - 1.0)
    inv = lax.rsqrt(var)
    return _apply(h2p, b2r, mu, inv)


# speedup vs baseline: 10.3333x; 1.2217x over previous
import functools

import jax
import jax.numpy as jnp
from jax import lax
from jax.experimental import pallas as pl
from jax.experimental.pallas import tpu as pltpu
from jax.experimental.pallas import tpu_sc as plsc

N_NODES = 100000
N_EDGES = 3200000
D_PAD = 32

NC = 2
NS = 16

R_PER_SC = N_NODES // NC
ACC_ROWS = 51200
CH = 128
BLK = 1024
N_CH = BLK // CH
NBLK = N_EDGES // BLK
ZCH = 128
OUT_CH = 1000
N_OUT_CHUNKS = R_PER_SC // OUT_CH


def _sc_aggregate(table_pad, src_idx, dst_idx):
    mesh = plsc.VectorSubcoreMesh(core_axis_name="c", subcore_axis_name="s")

    @functools.partial(
        pl.kernel,
        out_type=jax.ShapeDtypeStruct((N_NODES, D_PAD), jnp.float32),
        mesh=mesh,
        scratch_types=[
            pltpu.VMEM_SHARED((ACC_ROWS, D_PAD), jnp.float32),
            pltpu.VMEM((BLK,), jnp.int32),
            pltpu.VMEM((BLK,), jnp.int32),
            pltpu.VMEM((CH,), jnp.int32),
            pltpu.VMEM((CH,), jnp.int32),
            pltpu.VMEM((CH,), jnp.int32),
            pltpu.VMEM((CH,), jnp.int32),
            pltpu.VMEM((CH, D_PAD), jnp.float32),
            pltpu.VMEM((CH, D_PAD), jnp.float32),
            pltpu.SemaphoreType.DMA,
            pltpu.SemaphoreType.DMA,
            pltpu.SemaphoreType.DMA,
            pltpu.SemaphoreType.DMA,
        ],
        compiler_params=pltpu.CompilerParams(use_tc_tiling_on_sc=False),
    )
    def agg(table, sidx, didx, out, acc, src_big, dst_big,
            src0, src1, dst0, dst1, rows0, rows1, g0, g1, s0, s1):
        c = lax.axis_index("c")
        s = lax.axis_index("s")
        base = c * R_PER_SC
        srcs, dsts = [src0, src1], [dst0, dst1]
        rws, gsem, ssem = [rows0, rows1], [g0, g1], [s0, s1]

        zero = jnp.zeros((16,), jnp.float32)
        for r in range(ZCH):
            for q in range(D_PAD // 16):
                rows0[r, pl.ds(q * 16, 16)] = zero
        z0 = s * (ACC_ROWS // NS)

        def zloop(j, _):
            pltpu.sync_copy(rows0, acc.at[pl.ds(z0 + j * ZCH, ZCH)])
            return 0

        lax.fori_loop(0, (ACC_ROWS // NS) // ZCH, zloop, 0)
        plsc.subcore_barrier()

        def transform(b, sl):
            for v in range(CH // 16):
                t = dst_big[pl.ds(b * CH + v * 16, 16)]
                m = (t >= base) & (t < base + R_PER_SC)
                dsts[sl][pl.ds(v * 16, 16)] = jnp.where(m, t - base, R_PER_SC)
                srcs[sl][pl.ds(v * 16, 16)] = src_big[pl.ds(b * CH + v * 16, 16)]

        nblocks = (NBLK - s + NS - 1) // NS

        def outer(i, _):
            off = (s + i * NS) * BLK
            pltpu.sync_copy(sidx.at[pl.ds(off, BLK)], src_big)
            pltpu.sync_copy(didx.at[pl.ds(off, BLK)], dst_big)
            gd = [None, None]
            sd = [None, None]
            for b in range(N_CH):
                sl = b & 1
                if b >= 2:
                    sd[sl].wait()
                transform(b, sl)
                gd[sl] = pltpu.async_copy(table.at[srcs[sl]], rws[sl], gsem[sl])
                if b >= 1:
                    pv = 1 - sl
                    gd[pv].wait()
                    sd[pv] = pltpu.async_copy(
                        rws[pv], acc.at[dsts[pv]], ssem[pv], add=True
                    )
            last = (N_CH - 1) & 1
            gd[last].wait()
            sd[last] = pltpu.async_copy(
                rws[last], acc.at[dsts[last]], ssem[last], add=True
            )
            sd[1 - last].wait()
            sd[last].wait()
            return 0

        lax.fori_loop(0, nblocks, outer, 0)
        plsc.subcore_barrier()

        n_chunks = (N_OUT_CHUNKS - s + NS - 1) // NS

        def cloop(i, _):
            r0 = (s + i * NS) * OUT_CH
            pltpu.sync_copy(
                acc.at[pl.ds(r0, OUT_CH)], out.at[pl.ds(base + r0, OUT_CH)]
            )
            return 0

        lax.fori_loop(0, n_chunks, cloop, 0)

    return agg(table_pad, src_idx, dst_idx)


_BN = 2000
_NBLK = N_NODES // _BN


def _mlp_body(h_ref, w1_ref, b1_ref, w2_ref, o_ref):
    t = jnp.dot(h_ref[...], w1_ref[...], preferred_element_type=jnp.float32)
    t = jnp.maximum(t + b1_ref[...], 0.0)
    o_ref[...] = jnp.dot(t, w2_ref[...], preferred_element_type=jnp.float32)


def _mlp(h1p, W1p, b1r, W2p):
    return pl.pallas_call(
        _mlp_body,
        grid=(_NBLK,),
        in_specs=[
            pl.BlockSpec((_BN, D_PAD), lambda i: (i, 0)),
            pl.BlockSpec((D_PAD, 48), lambda i: (0, 0)),
            pl.BlockSpec((1, 48), lambda i: (0, 0)),
            pl.BlockSpec((48, D_PAD), lambda i: (0, 0)),
        ],
        out_specs=pl.BlockSpec((_BN, D_PAD), lambda i: (i, 0)),
        out_shape=jax.ShapeDtypeStruct((N_NODES, D_PAD), jnp.float32),
    )(h1p, W1p, b1r, W2p)


def _stats_body(h_ref, b2_ref, sum_ref, sq_ref):
    i = pl.program_id(0)
    x = h_ref[...] + b2_ref[...]
    col = lax.broadcasted_iota(jnp.int32, (_BN, D_PAD), 1)
    xs = jnp.where(col < 24, x, 0.0)

    @pl.when(i == 0)
    def _():
        sum_ref[...] = jnp.zeros((1, 1), jnp.float32)
        sq_ref[...] = jnp.zeros((1, 1), jnp.float32)

    sum_ref[...] += jnp.full((1, 1), jnp.sum(xs), jnp.float32)
    sq_ref[...] += jnp.full((1, 1), jnp.sum(xs * xs), jnp.float32)


def _stats(h2p, b2r):
    return pl.pallas_call(
        _stats_body,
        grid=(_NBLK,),
        in_specs=[
            pl.BlockSpec((_BN, D_PAD), lambda i: (i, 0)),
            pl.BlockSpec((1, D_PAD), lambda i: (0, 0)),
        ],
        out_specs=[
            pl.BlockSpec((1, 1), lambda i: (0, 0)),
            pl.BlockSpec((1, 1), lambda i: (0, 0)),
        ],
        out_shape=[
            jax.ShapeDtypeStruct((1, 1), jnp.float32),
            jax.ShapeDtypeStruct((1, 1), jnp.float32),
        ],
    )(h2p, b2r)


def _apply_body(h_ref, b2_ref, mu_ref, inv_ref, o_ref):
    x = h_ref[...][:, :24] + b2_ref[...][:, :24]
    o_ref[...] = (x - mu_ref[0, 0]) * inv_ref[0, 0]


def _apply(h2p, b2r, mu, inv):
    return pl.pallas_call(
        _apply_body,
        grid=(_NBLK,),
        in_specs=[
            pl.BlockSpec((_BN, D_PAD), lambda i: (i, 0)),
            pl.BlockSpec((1, D_PAD), lambda i: (0, 0)),
            pl.BlockSpec((1, 1), lambda i: (0, 0)),
            pl.BlockSpec((1, 1), lambda i: (0, 0)),
        ],
        out_specs=pl.BlockSpec((_BN, 24), lambda i: (i, 0)),
        out_shape=jax.ShapeDtypeStruct((N_NODES, 24), jnp.float32),
    )(h2p, b2r, mu, inv)


def kernel(features, edge_index, W1, b1, W2, b2):
    n, d = features.shape
    assert n == N_NODES and d == 24 and edge_index.shape == (2, N_EDGES)

    fpad = jnp.pad(features, ((0, 0), (0, D_PAD - 24)))
    W1p = jnp.pad(W1, ((0, D_PAD - 24), (0, 0)))
    W2p = jnp.pad(W2, ((0, 0), (0, D_PAD - 24)))
    b1r = b1.reshape(1, 48)
    b2r = jnp.pad(b2, (0, D_PAD - 24)).reshape(1, D_PAD)

    src_idx = edge_index[0]
    dst_idx = edge_index[1]
    h1p = _sc_aggregate(fpad, src_idx, dst_idx)
    yp = _mlp(h1p, W1p, b1r, W2p)
    h2p = _sc_aggregate(yp, src_idx, dst_idx)

    ssum, ssq = _stats(h2p, b2r)
    cnt = jnp.float32(N_NODES * 24)
    mu = ssum / cnt
    var = (ssq - cnt * mu * mu) / (cnt - 1.0)
    inv = lax.rsqrt(var)
    return _apply(h2p, b2r, mu, inv)


# speedup vs baseline: 25.9515x; 2.5114x over previous
import functools

import jax
import jax.numpy as jnp
from jax import lax
from jax.experimental import pallas as pl
from jax.experimental.pallas import tpu as pltpu
from jax.experimental.pallas import tpu_sc as plsc

N_NODES = 100000
N_EDGES = 3200000
D_PAD = 32
DH = 16

NC = 2
NS = 16

ACC_ROWS = 102400
CH = 128
N_CH = 8
BLK = CH * N_CH
NBLK = N_EDGES // BLK
N_IDX_ROWS = N_EDGES // CH
ZCH = 128
OUT_CH = 1000
N_OUT_CHUNKS = N_NODES // OUT_CH


def _sc_aggregate(table_flat, src2d, dst2d):
    mesh = plsc.VectorSubcoreMesh(core_axis_name="c", subcore_axis_name="s")

    rows_scr = [pltpu.VMEM((CH, DH), jnp.float32) for _ in range(N_CH)]
    sem_scr = [pltpu.SemaphoreType.DMA for _ in range(2 * N_CH)]

    @functools.partial(
        pl.kernel,
        out_type=jax.ShapeDtypeStruct((NC, N_NODES, DH), jnp.float32),
        mesh=mesh,
        scratch_types=[
            pltpu.VMEM_SHARED((ACC_ROWS, DH), jnp.float32),
            pltpu.VMEM((N_CH, CH), jnp.int32),
            pltpu.VMEM((N_CH, CH), jnp.int32),
        ] + rows_scr + sem_scr,
        compiler_params=pltpu.CompilerParams(use_tc_tiling_on_sc=False),
    )
    def agg(table, sidx, didx, out, acc, src_b, dst_b, *scr):
        rws = scr[:N_CH]
        gsem = scr[N_CH:2 * N_CH]
        ssem = scr[2 * N_CH:]
        c = lax.axis_index("c")
        s = lax.axis_index("s")

        zero = jnp.zeros((16,), jnp.float32)
        for r in range(ZCH):
            rws[0][r, pl.ds(0, 16)] = zero
        z0 = s * (ACC_ROWS // NS)

        def zloop(j, _):
            pltpu.sync_copy(rws[0], acc.at[pl.ds(z0 + j * ZCH, ZCH)])
            return 0

        lax.fori_loop(0, (ACC_ROWS // NS) // ZCH, zloop, 0)
        plsc.subcore_barrier()

        nblocks = (NBLK - s + NS - 1) // NS

        def outer(i, _):
            r0 = (s + i * NS) * N_CH
            pltpu.sync_copy(sidx.at[c, pl.ds(r0, N_CH)], src_b)
            pltpu.sync_copy(didx.at[pl.ds(r0, N_CH)], dst_b)
            gd = [None] * N_CH
            sd = [None] * N_CH
            for b in range(N_CH):
                gd[b] = pltpu.async_copy(
                    table.at[src_b.at[b]], rws[b], gsem[b]
                )
            for b in range(N_CH):
                gd[b].wait()
                sd[b] = pltpu.async_copy(
                    rws[b], acc.at[dst_b.at[b]], ssem[b], add=True
                )
            for b in range(N_CH):
                sd[b].wait()
            return 0

        lax.fori_loop(0, nblocks, outer, 0)
        plsc.subcore_barrier()

        n_chunks = (N_OUT_CHUNKS - s + NS - 1) // NS

        def cloop(i, _):
            r0 = (s + i * NS) * OUT_CH
            pltpu.sync_copy(
                acc.at[pl.ds(r0, OUT_CH)], out.at[c, pl.ds(r0, OUT_CH)]
            )
            return 0

        lax.fori_loop(0, n_chunks, cloop, 0)

    return agg(table_flat, src2d, dst2d)


_BN = 2000
_NBLK = N_NODES // _BN


def _mlp_body(h_ref, w1_ref, b1_ref, w2_ref, o_ref):
    h = jnp.concatenate([h_ref[0], h_ref[1]], axis=1)
    t = jnp.dot(h, w1_ref[...], preferred_element_type=jnp.float32)
    t = jnp.maximum(t + b1_ref[...], 0.0)
    y = jnp.dot(t, w2_ref[...], preferred_element_type=jnp.float32)
    o_ref[0] = y[:, :DH]
    o_ref[1] = y[:, DH:]


def _mlp(h1s, W1p, b1r, W2p):
    return pl.pallas_call(
        _mlp_body,
        grid=(_NBLK,),
        in_specs=[
            pl.BlockSpec((NC, _BN, DH), lambda i: (0, i, 0)),
            pl.BlockSpec((D_PAD, 48), lambda i: (0, 0)),
            pl.BlockSpec((1, 48), lambda i: (0, 0)),
            pl.BlockSpec((48, D_PAD), lambda i: (0, 0)),
        ],
        out_specs=pl.BlockSpec((NC, _BN, DH), lambda i: (0, i, 0)),
        out_shape=jax.ShapeDtypeStruct((NC, N_NODES, DH), jnp.float32),
    )(h1s, W1p, b1r, W2p)


def _stats_body(h_ref, b2_ref, sum_ref, sq_ref):
    i = pl.program_id(0)
    x = jnp.concatenate([h_ref[0], h_ref[1]], axis=1) + b2_ref[...]
    col = lax.broadcasted_iota(jnp.int32, (_BN, D_PAD), 1)
    xs = jnp.where(col < 24, x, 0.0)

    @pl.when(i == 0)
    def _():
        sum_ref[...] = jnp.zeros((1, 1), jnp.float32)
        sq_ref[...] = jnp.zeros((1, 1), jnp.float32)

    sum_ref[...] += jnp.full((1, 1), jnp.sum(xs), jnp.float32)
    sq_ref[...] += jnp.full((1, 1), jnp.sum(xs * xs), jnp.float32)


def _stats(h2s, b2r):
    return pl.pallas_call(
        _stats_body,
        grid=(_NBLK,),
        in_specs=[
            pl.BlockSpec((NC, _BN, DH), lambda i: (0, i, 0)),
            pl.BlockSpec((1, D_PAD), lambda i: (0, 0)),
        ],
        out_specs=[
            pl.BlockSpec((1, 1), lambda i: (0, 0)),
            pl.BlockSpec((1, 1), lambda i: (0, 0)),
        ],
        out_shape=[
            jax.ShapeDtypeStruct((1, 1), jnp.float32),
            jax.ShapeDtypeStruct((1, 1), jnp.float32),
        ],
    )(h2s, b2r)


def _apply_body(h_ref, b2_ref, mu_ref, inv_ref, o_ref):
    x = jnp.concatenate([h_ref[0], h_ref[1][:, :8]], axis=1)
    xb = x + b2_ref[...][:, :24]
    o_ref[...] = (xb - mu_ref[0, 0]) * inv_ref[0, 0]


def _apply(h2s, b2r, mu, inv):
    return pl.pallas_call(
        _apply_body,
        grid=(_NBLK,),
        in_specs=[
            pl.BlockSpec((NC, _BN, DH), lambda i: (0, i, 0)),
            pl.BlockSpec((1, D_PAD), lambda i: (0, 0)),
            pl.BlockSpec((1, 1), lambda i: (0, 0)),
            pl.BlockSpec((1, 1), lambda i: (0, 0)),
        ],
        out_specs=pl.BlockSpec((_BN, 24), lambda i: (i, 0)),
        out_shape=jax.ShapeDtypeStruct((N_NODES, 24), jnp.float32),
    )(h2s, b2r, mu, inv)


def kernel(features, edge_index, W1, b1, W2, b2):
    n, d = features.shape
    assert n == N_NODES and d == 24 and edge_index.shape == (2, N_EDGES)

    fpad = jnp.pad(features, ((0, 0), (0, D_PAD - 24)))
    ftab = jnp.concatenate([fpad[:, :DH], fpad[:, DH:]], axis=0)
    W1p = jnp.pad(W1, ((0, D_PAD - 24), (0, 0)))
    W2p = jnp.pad(W2, ((0, 0), (0, D_PAD - 24)))
    b1r = b1.reshape(1, 48)
    b2r = jnp.pad(b2, (0, D_PAD - 24)).reshape(1, D_PAD)

    src = edge_index[0]
    src2d = (src[None, :] + jnp.array([[0], [N_NODES]], jnp.int32)).reshape(
        NC, N_IDX_ROWS, CH
    )
    dst2d = edge_index[1].reshape(N_IDX_ROWS, CH)

    h1s = _sc_aggregate(ftab, src2d, dst2d)
    ys = _mlp(h1s, W1p, b1r, W2p)
    h2s = _sc_aggregate(ys.reshape(NC * N_NODES, DH), src2d, dst2d)

    ssum, ssq = _stats(h2s, b2r)
    cnt = jnp.float32(N_NODES * 24)
    mu = ssum / cnt
    var = (ssq - cnt * mu * mu) / (cnt - 1.0)
    inv = lax.rsqrt(var)
    return _apply(h2s, b2r, mu, inv)


# speedup vs baseline: 32.3246x; 1.2456x over previous
import functools

import jax
import jax.numpy as jnp
from jax import lax
from jax.experimental import pallas as pl
from jax.experimental.pallas import tpu as pltpu
from jax.experimental.pallas import tpu_sc as plsc

N_NODES = 100000
N_EDGES = 3200000
D_PAD = 32
DH = 16

NC = 2
NS = 16

ACC_ROWS = 102400
CH = 128
N_CH = 5
BLK = CH * N_CH
NBLK = N_EDGES // BLK
N_IDX_ROWS = N_EDGES // CH
ZCH = 128
OUT_CH = 1000
N_OUT_CHUNKS = N_NODES // OUT_CH


def _sc_aggregate(table_flat, src2d, dst2d):
    mesh = plsc.VectorSubcoreMesh(core_axis_name="c", subcore_axis_name="s")

    SCAT_BYTES = CH * DH * 4
    IDX_BYTES = 2 * N_CH * CH * 4

    @functools.partial(
        pl.kernel,
        out_type=jax.ShapeDtypeStruct((NC, N_NODES, DH), jnp.float32),
        mesh=mesh,
        scratch_types=[
            pltpu.VMEM_SHARED((ACC_ROWS, DH), jnp.float32),
            pltpu.VMEM((2, N_CH, CH), jnp.int32),
            pltpu.VMEM((2, N_CH, CH), jnp.int32),
            pltpu.VMEM((2, N_CH * CH, DH), jnp.float32),
            pltpu.SemaphoreType.DMA((2, N_CH)),
            pltpu.SemaphoreType.DMA((2, N_CH)),
            pltpu.SemaphoreType.DMA((2,)),
        ],
        compiler_params=pltpu.CompilerParams(use_tc_tiling_on_sc=False),
    )
    def agg(table, sidx, didx, out, acc, src_a, dst_a, rows_a, gsem, ssem, isem):
        c = lax.axis_index("c")
        s = lax.axis_index("s")

        zero = jnp.zeros((16,), jnp.float32)
        for r in range(ZCH):
            rows_a[0, r, pl.ds(0, 16)] = zero
        z0 = s * (ACC_ROWS // NS)

        def zloop(j, _):
            pltpu.sync_copy(
                rows_a.at[0, pl.ds(0, ZCH)], acc.at[pl.ds(z0 + j * ZCH, ZCH)]
            )
            return 0

        lax.fori_loop(0, (ACC_ROWS // NS) // ZCH, zloop, 0)
        plsc.subcore_barrier()

        nblocks = (NBLK - s + NS - 1) // NS

        pltpu.sync_copy(sidx.at[c, pl.ds(s * N_CH, N_CH)], src_a.at[0])
        pltpu.sync_copy(didx.at[pl.ds(s * N_CH, N_CH)], dst_a.at[0])

        def outer(i, _):
            p = i & 1
            q = 1 - p
            gd = []
            for b in range(N_CH):
                gd.append(pltpu.async_copy(
                    table.at[src_a.at[p, b]],
                    rows_a.at[p, pl.ds(b * CH, CH)],
                    gsem.at[p, b],
                ))

            @pl.when(i >= 1)
            def _():
                for b in range(N_CH):
                    pltpu.make_async_copy(
                        rows_a.at[q, pl.ds(b * CH, CH)],
                        acc.at[dst_a.at[q, b]],
                        ssem.at[q, b],
                    ).wait()

            @pl.when(i + 1 < nblocks)
            def _():
                r0n = (s + (i + 1) * NS) * N_CH
                pltpu.async_copy(
                    sidx.at[c, pl.ds(r0n, N_CH)], src_a.at[q], isem.at[q]
                )
                pltpu.async_copy(
                    didx.at[pl.ds(r0n, N_CH)], dst_a.at[q], isem.at[q]
                )

            for b in range(N_CH):
                gd[b].wait()
                pltpu.async_copy(
                    rows_a.at[p, pl.ds(b * CH, CH)],
                    acc.at[dst_a.at[p, b]],
                    ssem.at[p, b],
                    add=True,
                )

            @pl.when(i + 1 < nblocks)
            def _():
                r0n = (s + (i + 1) * NS) * N_CH
                pltpu.make_async_copy(
                    sidx.at[c, pl.ds(r0n, N_CH)], src_a.at[q], isem.at[q]
                ).wait()
                pltpu.make_async_copy(
                    didx.at[pl.ds(r0n, N_CH)], dst_a.at[q], isem.at[q]
                ).wait()

            return 0

        lax.fori_loop(0, nblocks, outer, 0)
        last = (nblocks - 1) & 1
        for b in range(N_CH):
            pltpu.make_async_copy(
                rows_a.at[last, pl.ds(b * CH, CH)],
                acc.at[dst_a.at[last, b]],
                ssem.at[last, b],
            ).wait()
        plsc.subcore_barrier()

        n_chunks = (N_OUT_CHUNKS - s + NS - 1) // NS

        def cloop(i, _):
            r0 = (s + i * NS) * OUT_CH
            pltpu.sync_copy(
                acc.at[pl.ds(r0, OUT_CH)], out.at[c, pl.ds(r0, OUT_CH)]
            )
            return 0

        lax.fori_loop(0, n_chunks, cloop, 0)

    return agg(table_flat, src2d, dst2d)


_BN = 2000
_NBLK = N_NODES // _BN


def _mlp_body(h_ref, w1_ref, b1_ref, w2_ref, o_ref):
    h = jnp.concatenate([h_ref[0], h_ref[1]], axis=1)
    t = jnp.dot(h, w1_ref[...], preferred_element_type=jnp.float32)
    t = jnp.maximum(t + b1_ref[...], 0.0)
    y = jnp.dot(t, w2_ref[...], preferred_element_type=jnp.float32)
    o_ref[0] = y[:, :DH]
    o_ref[1] = y[:, DH:]


def _mlp(h1s, W1p, b1r, W2p):
    return pl.pallas_call(
        _mlp_body,
        grid=(_NBLK,),
        in_specs=[
            pl.BlockSpec((NC, _BN, DH), lambda i: (0, i, 0)),
            pl.BlockSpec((D_PAD, 48), lambda i: (0, 0)),
            pl.BlockSpec((1, 48), lambda i: (0, 0)),
            pl.BlockSpec((48, D_PAD), lambda i: (0, 0)),
        ],
        out_specs=pl.BlockSpec((NC, _BN, DH), lambda i: (0, i, 0)),
        out_shape=jax.ShapeDtypeStruct((NC, N_NODES, DH), jnp.float32),
    )(h1s, W1p, b1r, W2p)


def _stats_body(h_ref, b2_ref, sum_ref, sq_ref):
    i = pl.program_id(0)
    x = jnp.concatenate([h_ref[0], h_ref[1]], axis=1) + b2_ref[...]
    col = lax.broadcasted_iota(jnp.int32, (_BN, D_PAD), 1)
    xs = jnp.where(col < 24, x, 0.0)

    @pl.when(i == 0)
    def _():
        sum_ref[...] = jnp.zeros((1, 1), jnp.float32)
        sq_ref[...] = jnp.zeros((1, 1), jnp.float32)

    sum_ref[...] += jnp.full((1, 1), jnp.sum(xs), jnp.float32)
    sq_ref[...] += jnp.full((1, 1), jnp.sum(xs * xs), jnp.float32)


def _stats(h2s, b2r):
    return pl.pallas_call(
        _stats_body,
        grid=(_NBLK,),
        in_specs=[
            pl.BlockSpec((NC, _BN, DH), lambda i: (0, i, 0)),
            pl.BlockSpec((1, D_PAD), lambda i: (0, 0)),
        ],
        out_specs=[
            pl.BlockSpec((1, 1), lambda i: (0, 0)),
            pl.BlockSpec((1, 1), lambda i: (0, 0)),
        ],
        out_shape=[
            jax.ShapeDtypeStruct((1, 1), jnp.float32),
            jax.ShapeDtypeStruct((1, 1), jnp.float32),
        ],
    )(h2s, b2r)


def _apply_body(h_ref, b2_ref, mu_ref, inv_ref, o_ref):
    x = jnp.concatenate([h_ref[0], h_ref[1][:, :8]], axis=1)
    xb = x + b2_ref[...][:, :24]
    o_ref[...] = (xb - mu_ref[0, 0]) * inv_ref[0, 0]


def _apply(h2s, b2r, mu, inv):
    return pl.pallas_call(
        _apply_body,
        grid=(_NBLK,),
        in_specs=[
            pl.BlockSpec((NC, _BN, DH), lambda i: (0, i, 0)),
            pl.BlockSpec((1, D_PAD), lambda i: (0, 0)),
            pl.BlockSpec((1, 1), lambda i: (0, 0)),
            pl.BlockSpec((1, 1), lambda i: (0, 0)),
        ],
        out_specs=pl.BlockSpec((_BN, 24), lambda i: (i, 0)),
        out_shape=jax.ShapeDtypeStruct((N_NODES, 24), jnp.float32),
    )(h2s, b2r, mu, inv)


def kernel(features, edge_index, W1, b1, W2, b2):
    n, d = features.shape
    assert n == N_NODES and d == 24 and edge_index.shape == (2, N_EDGES)

    fpad = jnp.pad(features, ((0, 0), (0, D_PAD - 24)))
    ftab = jnp.concatenate([fpad[:, :DH], fpad[:, DH:]], axis=0)
    W1p = jnp.pad(W1, ((0, D_PAD - 24), (0, 0)))
    W2p = jnp.pad(W2, ((0, 0), (0, D_PAD - 24)))
    b1r = b1.reshape(1, 48)
    b2r = jnp.pad(b2, (0, D_PAD - 24)).reshape(1, D_PAD)

    src = edge_index[0]
    src2d = (src[None, :] + jnp.array([[0], [N_NODES]], jnp.int32)).reshape(
        NC, N_IDX_ROWS, CH
    )
    dst2d = edge_index[1].reshape(N_IDX_ROWS, CH)

    h1s = _sc_aggregate(ftab, src2d, dst2d)
    ys = _mlp(h1s, W1p, b1r, W2p)
    h2s = _sc_aggregate(ys.reshape(NC * N_NODES, DH), src2d, dst2d)

    ssum, ssq = _stats(h2s, b2r)
    cnt = jnp.float32(N_NODES * 24)
    mu = ssum / cnt
    var = (ssq - cnt * mu * mu) / (cnt - 1.0)
    inv = lax.rsqrt(var)
    return _apply(h2s, b2r, mu, inv)
